# Initial kernel scaffold; baseline (speedup 1.0000x reference)
import jax
import jax.numpy as jnp
from jax.experimental import pallas as pl


def kernel(seq_poi_embeddings, hour_set, isweekend_set, user_set, hour_table, isweekend_table, user_table):
    raise NotImplementedError("write your pallas kernel here")


# speedup vs baseline: 1.4507x; 1.4507x over previous
import functools

import jax
import jax.numpy as jnp
from jax import lax
from jax.experimental import pallas as pl
from jax.experimental.pallas import tpu as pltpu
from jax.experimental.pallas import tpu_sc as plsc

B, L = 4096, 200
N = B * L
POI_DIM = 128
HOUR_DIM = 16
WKND_DIM = 8
USER_DIM = 64
OUT_DIM = POI_DIM + HOUR_DIM + WKND_DIM + USER_DIM

NUM_CORES = 2
NUM_SUBCORES = 16
NW = NUM_CORES * NUM_SUBCORES
TOK_PER_W = N // NW
CHUNK = 512
IDX_W = 128
NIDX = CHUNK // IDX_W
NCHUNK = TOK_PER_W // CHUNK


def _emb_body(poi_hbm, hour_hbm, wknd_hbm, user_hbm,
              hour_tbl, wknd_tbl, user_tbl, out_hbm,
              h_idx, w_idx, u_idx, p_rows, h_rows, w_rows, u_rows, t_rows,
              sem):
    wid = lax.axis_index("s") * NUM_CORES + lax.axis_index("c")
    w_base = wid * TOK_PER_W

    def chunk_body(i):
        base = w_base + i * CHUNK
        ibase = wid * (TOK_PER_W // IDX_W) + i * NIDX
        pltpu.sync_copy(hour_hbm.at[pl.ds(ibase, NIDX)], h_idx)
        pltpu.sync_copy(wknd_hbm.at[pl.ds(ibase, NIDX)], w_idx)
        pltpu.sync_copy(user_hbm.at[pl.ds(ibase, NIDX)], u_idx)
        cps = [pltpu.async_copy(poi_hbm.at[pl.ds(base, CHUNK)], p_rows, sem)]
        for j in range(NIDX):
            sl = pl.ds(j * IDX_W, IDX_W)
            cps.append(pltpu.async_copy(
                hour_tbl.at[h_idx.at[j]], h_rows.at[sl], sem))
            cps.append(pltpu.async_copy(
                wknd_tbl.at[w_idx.at[j]], w_rows.at[sl], sem))
            cps.append(pltpu.async_copy(
                user_tbl.at[u_idx.at[j]], u_rows.at[sl], sem))
        for cp in cps:
            cp.wait()
        out = out_hbm.at[pl.ds(base, CHUNK)]
        pltpu.sync_copy(p_rows, out.at[:, pl.ds(0, POI_DIM)])
        pltpu.sync_copy(h_rows, out.at[:, pl.ds(POI_DIM, HOUR_DIM)])
        pltpu.sync_copy(w_rows, out.at[:, pl.ds(POI_DIM + HOUR_DIM, WKND_DIM)])
        pltpu.sync_copy(u_rows, out.at[:, pl.ds(OUT_DIM - USER_DIM, USER_DIM)])

    pl.loop(0, NCHUNK)(chunk_body)


_mesh = plsc.VectorSubcoreMesh(core_axis_name="c", subcore_axis_name="s")

_emb_kernel = functools.partial(
    pl.kernel,
    out_type=jax.ShapeDtypeStruct((N, OUT_DIM), jnp.float32),
    mesh=_mesh,
    compiler_params=pltpu.CompilerParams(use_tc_tiling_on_sc=False),
    scratch_types=[
        pltpu.VMEM((NIDX, IDX_W), jnp.int32),
        pltpu.VMEM((NIDX, IDX_W), jnp.int32),
        pltpu.VMEM((NIDX, IDX_W), jnp.int32),
        pltpu.VMEM((CHUNK, POI_DIM), jnp.float32),
        pltpu.VMEM((CHUNK, HOUR_DIM), jnp.float32),
        pltpu.VMEM((CHUNK, WKND_DIM), jnp.float32),
        pltpu.VMEM((CHUNK, USER_DIM), jnp.float32),
        pltpu.VMEM((CHUNK, OUT_DIM - POI_DIM), jnp.float32),
        pltpu.SemaphoreType.DMA,
    ],
)(_emb_body)


@jax.jit
def kernel(seq_poi_embeddings, hour_set, isweekend_set, user_set,
           hour_table, isweekend_table, user_table):
    poi = seq_poi_embeddings.reshape(N, POI_DIM)
    hour = hour_set.reshape(N // IDX_W, IDX_W)
    wknd = isweekend_set.reshape(N // IDX_W, IDX_W)
    user = user_set.reshape(N // IDX_W, IDX_W)
    h_tbl = hour_table.at[0].set(0.0)
    w_tbl = isweekend_table.at[0].set(0.0)
    u_tbl = user_table.at[0].set(0.0)
    out = _emb_kernel(poi, hour, wknd, user, h_tbl, w_tbl, u_tbl)
    return out.reshape(B, L, OUT_DIM)


# speedup vs baseline: 2.9614x; 2.0413x over previous
import functools

import jax
import jax.numpy as jnp
from jax import lax
from jax.experimental import pallas as pl
from jax.experimental.pallas import tpu as pltpu
from jax.experimental.pallas import tpu_sc as plsc

B, L = 4096, 200
N = B * L
POI_DIM = 128
HOUR_DIM = 16
WKND_DIM = 8
USER_DIM = 64
HW_DIM = HOUR_DIM + WKND_DIM
TAIL_DIM = HW_DIM + USER_DIM
OUT_DIM = POI_DIM + TAIL_DIM

NUM_CORES = 2
NUM_SUBCORES = 16
NW = NUM_CORES * NUM_SUBCORES
TOK_PER_W = N // NW
CHUNK = 256
IDX_W = 128
NIDX = CHUNK // IDX_W
NCHUNK = TOK_PER_W // CHUNK


def _emb_body(poi_hbm, hour_hbm, wknd_hbm, user_hbm,
              hw_tbl, u_tbl, out_hbm,
              h_idx, w_idx, u_idx, hw_idx, hw_rows, t_rows, p_rows, sem,
              sem_w):
    wid = lax.axis_index("s") * NUM_CORES + lax.axis_index("c")
    w_base = wid * TOK_PER_W

    def chunk_body(i):
        base = w_base + i * CHUNK
        ibase = wid * (TOK_PER_W // IDX_W) + i * NIDX
        cps = [
            pltpu.async_copy(hour_hbm.at[pl.ds(ibase, NIDX)], h_idx, sem),
            pltpu.async_copy(wknd_hbm.at[pl.ds(ibase, NIDX)], w_idx, sem),
            pltpu.async_copy(user_hbm.at[pl.ds(ibase, NIDX)], u_idx, sem),
        ]
        cp_p = pltpu.async_copy(poi_hbm.at[pl.ds(base, CHUNK)], p_rows, sem_w)
        for cp in cps:
            cp.wait()
        for j in range(NIDX):
            for k in range(IDX_W // 16):
                sl = pl.ds(k * 16, 16)
                hw_idx[j, sl] = h_idx[j, sl] * 3 + w_idx[j, sl]
        cps = []
        for j in range(NIDX):
            sl = pl.ds(j * IDX_W, IDX_W)
            cps.append(pltpu.async_copy(
                u_tbl.at[u_idx.at[j]], t_rows.at[sl], sem))
            cps.append(pltpu.async_copy(
                hw_tbl.at[hw_idx.at[j]], hw_rows.at[sl], sem))
        for cp in cps:
            cp.wait()

        def tok_body(t):
            t_rows[t, pl.ds(0, 16)] = hw_rows[t, pl.ds(0, 16)]
            t_rows[t, pl.ds(8, 16)] = hw_rows[t, pl.ds(8, 16)]
        pl.loop(0, CHUNK, unroll=8)(tok_body)

        cp_p.wait()
        out = out_hbm.at[pl.ds(base, CHUNK)]
        cp1 = pltpu.async_copy(p_rows, out.at[:, pl.ds(0, POI_DIM)], sem_w)
        cp2 = pltpu.async_copy(t_rows, out.at[:, pl.ds(POI_DIM, TAIL_DIM)],
                               sem_w)
        cp1.wait()
        cp2.wait()

    pl.loop(0, NCHUNK)(chunk_body)


_mesh = plsc.VectorSubcoreMesh(core_axis_name="c", subcore_axis_name="s")

_emb_kernel = functools.partial(
    pl.kernel,
    out_type=jax.ShapeDtypeStruct((N, OUT_DIM), jnp.float32),
    mesh=_mesh,
    compiler_params=pltpu.CompilerParams(use_tc_tiling_on_sc=False),
    scratch_types=[
        pltpu.VMEM((NIDX, IDX_W), jnp.int32),
        pltpu.VMEM((NIDX, IDX_W), jnp.int32),
        pltpu.VMEM((NIDX, IDX_W), jnp.int32),
        pltpu.VMEM((NIDX, IDX_W), jnp.int32),
        pltpu.VMEM((CHUNK, HW_DIM), jnp.float32),
        pltpu.VMEM((CHUNK, TAIL_DIM), jnp.float32),
        pltpu.VMEM((CHUNK, POI_DIM), jnp.float32),
        pltpu.SemaphoreType.DMA,
        pltpu.SemaphoreType.DMA,
    ],
)(_emb_body)


@jax.jit
def kernel(seq_poi_embeddings, hour_set, isweekend_set, user_set,
           hour_table, isweekend_table, user_table):
    poi = seq_poi_embeddings.reshape(N, POI_DIM)
    hour = hour_set.reshape(N // IDX_W, IDX_W)
    wknd = isweekend_set.reshape(N // IDX_W, IDX_W)
    user = user_set.reshape(N // IDX_W, IDX_W)
    h_tbl = hour_table.at[0].set(0.0)
    w_tbl = isweekend_table.at[0].set(0.0)
    hw_tbl = jnp.concatenate(
        [jnp.broadcast_to(h_tbl[:, None, :], (25, 3, HOUR_DIM)),
         jnp.broadcast_to(w_tbl[None, :, :], (25, 3, WKND_DIM))],
        axis=2).reshape(75, HW_DIM)
    u_tbl = jnp.concatenate(
        [jnp.zeros((100001, HW_DIM), jnp.float32),
         user_table.at[0].set(0.0)], axis=1)
    out = _emb_kernel(poi, hour, wknd, user, hw_tbl, u_tbl)
    return out.reshape(B, L, OUT_DIM)


# speedup vs baseline: 2.9651x; 1.0013x over previous
import functools

import jax
import jax.numpy as jnp
from jax import lax
from jax.experimental import pallas as pl
from jax.experimental.pallas import tpu as pltpu
from jax.experimental.pallas import tpu_sc as plsc

B, L = 4096, 200
N = B * L
POI_DIM = 128
HOUR_DIM = 16
WKND_DIM = 8
USER_DIM = 64
HW_DIM = HOUR_DIM + WKND_DIM
TAIL_DIM = HW_DIM + USER_DIM
OUT_DIM = POI_DIM + TAIL_DIM

NUM_CORES = 2
NUM_SUBCORES = 16
NW = NUM_CORES * NUM_SUBCORES
ROWS_PER_W = B // NW
CB = 2
CHUNK = CB * L
NCHUNK = ROWS_PER_W // CB
IDX_GROUPS = ((0, 128), (128, 128), (256, 128), (384, 16))


def _emb_body(poi_hbm, hour_hbm, wknd_hbm, user_hbm,
              hw_tbl, u_tbl, out_hbm,
              h_idx, w_idx, u_idx, hw_idx, hw_rows, t_rows, p_rows, sem,
              sem_w):
    wid = lax.axis_index("s") * NUM_CORES + lax.axis_index("c")
    w_base = wid * ROWS_PER_W

    def chunk_body(i):
        b0 = w_base + i * CB
        base = b0 * L
        tsl = pl.ds(base, CHUNK)
        cps = [
            pltpu.async_copy(hour_hbm.at[tsl], h_idx, sem),
            pltpu.async_copy(wknd_hbm.at[tsl], w_idx, sem),
            pltpu.async_copy(user_hbm.at[tsl], u_idx, sem),
        ]
        cps_p = [
            pltpu.async_copy(poi_hbm.at[b0 + bl],
                             p_rows.at[pl.ds(bl * L, L)], sem_w)
            for bl in range(CB)
        ]
        for cp in cps:
            cp.wait()
        for k in range(CHUNK // 16):
            sl = pl.ds(k * 16, 16)
            hw_idx[sl] = h_idx[sl] * 3 + w_idx[sl]
        cps = []
        for (off, ln) in IDX_GROUPS:
            d = pl.ds(off, ln)
            cps.append(pltpu.async_copy(
                u_tbl.at[u_idx.at[d]], t_rows.at[d], sem))
            cps.append(pltpu.async_copy(
                hw_tbl.at[hw_idx.at[d]], hw_rows.at[d], sem))
        for cp in cps:
            cp.wait()

        def tok_body(t):
            t_rows[t, pl.ds(0, 16)] = hw_rows[t, pl.ds(0, 16)]
            t_rows[t, pl.ds(8, 16)] = hw_rows[t, pl.ds(8, 16)]
        pl.loop(0, CHUNK, unroll=8)(tok_body)

        for cp in cps_p:
            cp.wait()
        out = out_hbm.at[pl.ds(base, CHUNK)]
        cp1 = pltpu.async_copy(p_rows, out.at[:, pl.ds(0, POI_DIM)], sem_w)
        cp2 = pltpu.async_copy(t_rows, out.at[:, pl.ds(POI_DIM, TAIL_DIM)],
                               sem_w)
        cp1.wait()
        cp2.wait()

    pl.loop(0, NCHUNK)(chunk_body)


_mesh = plsc.VectorSubcoreMesh(core_axis_name="c", subcore_axis_name="s")

_emb_kernel = functools.partial(
    pl.kernel,
    out_type=jax.ShapeDtypeStruct((N, OUT_DIM), jnp.float32),
    mesh=_mesh,
    compiler_params=pltpu.CompilerParams(use_tc_tiling_on_sc=False),
    scratch_types=[
        pltpu.VMEM((CHUNK,), jnp.int32),
        pltpu.VMEM((CHUNK,), jnp.int32),
        pltpu.VMEM((CHUNK,), jnp.int32),
        pltpu.VMEM((CHUNK,), jnp.int32),
        pltpu.VMEM((CHUNK, HW_DIM), jnp.float32),
        pltpu.VMEM((CHUNK, TAIL_DIM), jnp.float32),
        pltpu.VMEM((CHUNK, POI_DIM), jnp.float32),
        pltpu.SemaphoreType.DMA,
        pltpu.SemaphoreType.DMA,
    ],
)(_emb_body)


@jax.jit
def kernel(seq_poi_embeddings, hour_set, isweekend_set, user_set,
           hour_table, isweekend_table, user_table):
    hour = hour_set.reshape(N)
    wknd = isweekend_set.reshape(N)
    user = user_set.reshape(N)
    h_tbl = hour_table.at[0].set(0.0)
    w_tbl = isweekend_table.at[0].set(0.0)
    hw_tbl = jnp.concatenate(
        [jnp.broadcast_to(h_tbl[:, None, :], (25, 3, HOUR_DIM)),
         jnp.broadcast_to(w_tbl[None, :, :], (25, 3, WKND_DIM))],
        axis=2).reshape(75, HW_DIM)
    u_tbl = jnp.concatenate(
        [jnp.zeros((100001, HW_DIM), jnp.float32),
         user_table.at[0].set(0.0)], axis=1)
    out = _emb_kernel(seq_poi_embeddings, hour, wknd, user, hw_tbl, u_tbl)
    return out.reshape(B, L, OUT_DIM)
